# baseline (speedup 1.0000x reference)
import functools

import jax
import jax.numpy as jnp
from jax import lax
from jax.experimental import pallas as pl
from jax.experimental.pallas import tpu as pltpu
from jax.experimental.pallas import tpu_sc as plsc

B = 256
P = 1024
F = 64
D = 64
C = 16
N = 64
K = F * D


def _sc_gather(table, idx):
    info = plsc.get_sparse_core_info()
    nw = info.num_cores * info.num_subcores
    rows = B * F
    b_per_w = rows // nw
    mesh = plsc.VectorSubcoreMesh(core_axis_name="c", subcore_axis_name="s")

    @functools.partial(
        pl.kernel,
        mesh=mesh,
        out_type=jax.ShapeDtypeStruct((rows, 2 * D), jnp.float32),
        scratch_types=[
            pltpu.VMEM((b_per_w,), jnp.int32),
            pltpu.VMEM((b_per_w, 2 * D), jnp.float32),
            pltpu.SemaphoreType.DMA,
        ],
    )
    def gather_k(tab_hbm, idx_hbm, out_hbm, idx_v, rows_v, sem):
        wid = lax.axis_index("s") * info.num_cores + lax.axis_index("c")
        base = wid * b_per_w
        pltpu.sync_copy(idx_hbm.at[pl.ds(base, b_per_w)], idx_v)
        pltpu.async_copy(tab_hbm.at[idx_v], rows_v, sem).wait()
        pltpu.sync_copy(rows_v, out_hbm.at[pl.ds(base, b_per_w)])

    return gather_k(table, idx)


def _mm_body(xg_ref, w_ref, y_ref, s_ref):
    j = pl.program_id(0)
    c = pl.program_id(1)
    xb = xg_ref[...].reshape(B, 2 * D)
    w = w_ref[0, 0]
    wpad = jnp.concatenate([w, jnp.zeros((D, N), jnp.float32)], axis=0)
    contrib = jnp.dot(xb, wpad, preferred_element_type=jnp.float32)

    @pl.when(j == 0)
    def _():
        y_ref[c] = contrib

    @pl.when(j > 0)
    def _():
        y_ref[c] += contrib

    @pl.when(c == 0)
    def _():
        lane = lax.broadcasted_iota(jnp.int32, (B, 2 * D), 1)
        ps = jnp.sum(jnp.where(lane < D, xb, 0.0), axis=1)

        @pl.when(j == 0)
        def _():
            s_ref[0] = ps

        @pl.when(j > 0)
        def _():
            s_ref[0] += ps


def _matmul(xg4, w4):
    return pl.pallas_call(
        _mm_body,
        grid=(F, C),
        in_specs=[
            pl.BlockSpec((B, 1, 1, 2 * D), lambda j, c: (0, j, 0, 0)),
            pl.BlockSpec((1, 1, D, N), lambda j, c: (c, j, 0, 0)),
        ],
        out_specs=[
            pl.BlockSpec((C, B, N), lambda j, c: (0, 0, 0)),
            pl.BlockSpec((1, B), lambda j, c: (0, 0)),
        ],
        out_shape=[
            jax.ShapeDtypeStruct((C, B, N), jnp.float32),
            jax.ShapeDtypeStruct((1, B), jnp.float32),
        ],
    )(xg4, w4)


def _sample_body(y_ref, s_ref, g_ref, o_ref):
    y = y_ref[...]
    s = s_ref[...]
    sinv = jnp.where(s > 0, 1.0 / s, 0.0)
    fam = jnp.max(y, axis=2)
    avg = jnp.mean(fam * sinv)
    temp = 1.0 / (avg + 0.0001) - 1.0
    scale = (sinv / temp).reshape(1, B, 1)
    z = y * scale + g_ref[...]
    m = jnp.max(z, axis=2, keepdims=True)
    iota = lax.broadcasted_iota(jnp.int32, (C, B, N), 2)
    kidx = jnp.min(jnp.where(z == m, iota, N), axis=2, keepdims=True)
    o_ref[...] = (iota == kidx).astype(jnp.float32)


def _sample(y, s, g):
    return pl.pallas_call(
        _sample_body,
        out_shape=jax.ShapeDtypeStruct((C, B, N), jnp.float32),
    )(y, s, g)


def kernel(x, weights, input_filter):
    table = x.reshape(B * P // 2, 2 * D)
    idx = (jnp.arange(B, dtype=jnp.int32)[:, None] * (P // 2)
           + (input_filter[None, :].astype(jnp.int32) // 2)).reshape(-1)
    xg2 = _sc_gather(table, idx)
    y, s = _matmul(xg2.reshape(B, F, 1, 2 * D),
                   weights.reshape(C, F, D, N))
    gum = jnp.transpose(
        jax.random.gumbel(jax.random.key(42), (B, C, N), jnp.float32), (1, 0, 2))
    oh = _sample(y, s, gum)
    return jnp.transpose(oh, (1, 0, 2))


# speedup vs baseline: 8.7304x; 8.7304x over previous
import jax
import jax.numpy as jnp
from jax import lax
from jax.experimental import pallas as pl

B = 256
P = 1024
F = 64
D = 64
C = 16
N = 64
K = F * D
GB = 16


def _ka_body(filt_ref, x_ref, o_ref):
    xb = x_ref[...]
    fv = filt_ref[0]
    pio = lax.broadcasted_iota(jnp.int32, (P, F), 0)
    sel = (pio == fv).astype(jnp.float32)
    r = jnp.dot(xb, sel, preferred_element_type=jnp.float32,
                precision=lax.Precision.HIGHEST)
    o_ref[...] = r.reshape(GB, D, F)


def _ka(filt3, x2d):
    return pl.pallas_call(
        _ka_body,
        grid=(B // GB,),
        in_specs=[
            pl.BlockSpec((1, 1, F), lambda g: (0, 0, 0)),
            pl.BlockSpec((GB * D, P), lambda g: (g, 0)),
        ],
        out_specs=pl.BlockSpec((GB, D, F), lambda g: (g, 0, 0)),
        out_shape=jax.ShapeDtypeStruct((B, D, F), jnp.float32),
    )(filt3, x2d)


def _kb_body(wt_ref, xgt_ref, y_ref, s_ref):
    c = pl.program_id(0)
    xgt = xgt_ref[...]
    y_ref[0] = jnp.dot(wt_ref[0], xgt, preferred_element_type=jnp.float32,
                       precision=lax.Precision.HIGHEST)

    @pl.when(c == 0)
    def _():
        s_ref[0] = jnp.sum(xgt, axis=0)


def _kb(wt, xgt):
    return pl.pallas_call(
        _kb_body,
        grid=(C,),
        in_specs=[
            pl.BlockSpec((1, N, K), lambda c: (c, 0, 0)),
            pl.BlockSpec((K, B), lambda c: (0, 0)),
        ],
        out_specs=[
            pl.BlockSpec((1, N, B), lambda c: (c, 0, 0)),
            pl.BlockSpec((1, B), lambda c: (0, 0)),
        ],
        out_shape=[
            jax.ShapeDtypeStruct((C, N, B), jnp.float32),
            jax.ShapeDtypeStruct((1, B), jnp.float32),
        ],
    )(wt, xgt)


def _kc_body(y_ref, s_ref, g_ref, o_ref):
    y = y_ref[...]
    s = s_ref[...]
    sinv = jnp.where(s > 0, 1.0 / s, 0.0)
    fam = jnp.max(y, axis=1)
    avg = jnp.mean(fam * sinv)
    temp = 1.0 / (avg + 0.0001) - 1.0
    scale = (sinv / temp).reshape(1, 1, B)
    z = y * scale + g_ref[...]
    m = jnp.max(z, axis=1, keepdims=True)
    iota = lax.broadcasted_iota(jnp.int32, (C, N, B), 1)
    kidx = jnp.min(jnp.where(z == m, iota, N), axis=1, keepdims=True)
    o_ref[...] = (iota == kidx).astype(jnp.float32)


def _kc(y, s, g):
    return pl.pallas_call(
        _kc_body,
        out_shape=jax.ShapeDtypeStruct((C, N, B), jnp.float32),
    )(y, s, g)


def kernel(x, weights, input_filter):
    x2d = jnp.transpose(x, (0, 2, 3, 1)).reshape(B * D, P)
    wt = jnp.transpose(weights, (0, 2, 1))
    filt3 = input_filter.astype(jnp.int32).reshape(1, 1, F)

    xsel = _ka(filt3, x2d)
    xgt = jnp.transpose(xsel, (2, 1, 0)).reshape(K, B)
    yt, s = _kb(wt, xgt)

    gum = jnp.transpose(
        jax.random.gumbel(jax.random.key(42), (B, C, N), jnp.float32),
        (1, 2, 0))
    oh = _kc(yt, s, gum)
    return jnp.transpose(oh, (2, 0, 1))


# speedup vs baseline: 11.1225x; 1.2740x over previous
import jax
import jax.numpy as jnp
from jax import lax
from jax.experimental import pallas as pl
from jax.experimental.pallas import tpu as pltpu

B = 256
P = 1024
F = 64
D = 64
C = 16
N = 64
K = F * D
GB = 16


def _ka_body(filt_ref, x_ref, o_ref):
    xb = x_ref[...]
    fv = filt_ref[0]
    pio = lax.broadcasted_iota(jnp.int32, (P, F), 0)
    sel = (pio == fv).astype(jnp.float32)
    r = jnp.dot(xb, sel, preferred_element_type=jnp.float32,
                precision=lax.Precision.HIGHEST)
    o_ref[...] = r.reshape(GB, D, F)


def _ka(filt3, x2d):
    return pl.pallas_call(
        _ka_body,
        grid=(B // GB,),
        in_specs=[
            pl.BlockSpec((1, 1, F), lambda g: (0, 0, 0)),
            pl.BlockSpec((GB * D, P), lambda g: (g, 0)),
        ],
        out_specs=pl.BlockSpec((GB, D, F), lambda g: (g, 0, 0)),
        out_shape=jax.ShapeDtypeStruct((B, D, F), jnp.float32),
    )(filt3, x2d)


def _kb_body(wt_ref, xgt_ref, g_ref, o_ref, xh_ref, xl_ref, y_ref, s_ref):
    c = pl.program_id(0)

    @pl.when(c == 0)
    def _():
        xg = xgt_ref[...]
        hi = xg.astype(jnp.bfloat16)
        xh_ref[...] = hi
        xl_ref[...] = (xg - hi.astype(jnp.float32)).astype(jnp.bfloat16)
        s_ref[0] = jnp.sum(xg, axis=0)

    wh = wt_ref[0].astype(jnp.bfloat16)
    y_ref[c] = (
        jnp.dot(wh, xh_ref[...], preferred_element_type=jnp.float32)
        + jnp.dot(wh, xl_ref[...], preferred_element_type=jnp.float32))

    @pl.when(c == C - 1)
    def _():
        y = y_ref[...]
        s = s_ref[...]
        sinv = jnp.where(s > 0, 1.0 / s, 0.0)
        fam = jnp.max(y, axis=1)
        avg = jnp.mean(fam * sinv)
        temp = 1.0 / (avg + 0.0001) - 1.0
        scale = (sinv / temp).reshape(1, 1, B)
        z = y * scale + g_ref[...]
        m = jnp.max(z, axis=1, keepdims=True)
        iota = lax.broadcasted_iota(jnp.int32, (C, N, B), 1)
        kidx = jnp.min(jnp.where(z == m, iota, N), axis=1, keepdims=True)
        o_ref[...] = (iota == kidx).astype(jnp.float32)


def _kb(wt, xgt, gum):
    return pl.pallas_call(
        _kb_body,
        grid=(C,),
        in_specs=[
            pl.BlockSpec((1, N, K), lambda c: (c, 0, 0)),
            pl.BlockSpec((K, B), lambda c: (0, 0)),
            pl.BlockSpec((C, N, B), lambda c: (0, 0, 0)),
        ],
        out_specs=pl.BlockSpec((C, N, B), lambda c: (0, 0, 0)),
        out_shape=jax.ShapeDtypeStruct((C, N, B), jnp.float32),
        scratch_shapes=[
            pltpu.VMEM((K, B), jnp.bfloat16),
            pltpu.VMEM((K, B), jnp.bfloat16),
            pltpu.VMEM((C, N, B), jnp.float32),
            pltpu.VMEM((1, B), jnp.float32),
        ],
    )(wt, xgt, gum)


def kernel(x, weights, input_filter):
    x2d = jnp.transpose(x, (0, 2, 3, 1)).reshape(B * D, P)
    wt = jnp.transpose(weights, (0, 2, 1))
    filt3 = input_filter.astype(jnp.int32).reshape(1, 1, F)

    xsel = _ka(filt3, x2d)
    xgt = jnp.transpose(xsel, (2, 1, 0)).reshape(K, B)

    gum = jnp.transpose(
        jax.random.gumbel(jax.random.key(42), (B, C, N), jnp.float32),
        (1, 2, 0))
    oh = _kb(wt, xgt, gum)
    return jnp.transpose(oh, (2, 0, 1))
